# baseline (speedup 1.0000x reference)
import functools

import jax
import jax.numpy as jnp
from jax import lax
from jax.experimental import pallas as pl
from jax.experimental.pallas import tpu as pltpu
from jax.experimental.pallas import tpu_sc as plsc

N = 10000
E = 320000
B = 256
NPAD = 10240
EPAD = 327680
D1P = 128
D2P = 32
EROWS = EPAD // 128
ROWS_PER_W = EROWS // 32
CHUNK_ROWS = 2
NCHUNK = ROWS_PER_W // CHUNK_ROWS
TILE_ROWS = NPAD // 16


def _mm1_body(x_ref, w_ref, o_ref):
    y = jnp.dot(x_ref[...], w_ref[...], preferred_element_type=jnp.float32)
    lane = lax.broadcasted_iota(jnp.int32, y.shape, 1)
    o_ref[...] = (y + jnp.where(lane == 100, 1.0, 0.0)).astype(jnp.bfloat16)


def _mm1(x, w1p):
    return pl.pallas_call(
        _mm1_body,
        grid=(10,),
        in_specs=[pl.BlockSpec((1000, 128), lambda i: (i, 0)),
                  pl.BlockSpec((128, D1P), lambda i: (0, 0))],
        out_specs=pl.BlockSpec((1000, D1P), lambda i: (i, 0)),
        out_shape=jax.ShapeDtypeStruct((N, D1P), jnp.bfloat16),
    )(x, w1p)


def _make_agg(d, cr, n0, n1):
    assert (n0 + n1) * 16 * cr == EROWS and n0 % 2 == 0 and n1 % 2 == 0
    mesh = plsc.VectorSubcoreMesh(core_axis_name="c", subcore_axis_name="s")

    @functools.partial(
        pl.kernel,
        mesh=mesh,
        compiler_params=pltpu.CompilerParams(use_tc_tiling_on_sc=False),
        out_type=jax.ShapeDtypeStruct((2 * NPAD, d), jnp.bfloat16),
        scratch_types=[
            pltpu.VMEM((cr, 128), jnp.int32),
            pltpu.VMEM((cr, 128), jnp.int32),
            pltpu.VMEM((cr * 128, d), jnp.bfloat16),
            pltpu.VMEM((cr, 128), jnp.int32),
            pltpu.VMEM((cr, 128), jnp.int32),
            pltpu.VMEM((cr * 128, d), jnp.bfloat16),
            pltpu.VMEM_SHARED((NPAD, d), jnp.bfloat16),
            pltpu.SemaphoreType.DMA,
            pltpu.SemaphoreType.DMA,
        ],
    )
    def agg(y_hbm, src_hbm, dst_hbm, out_hbm,
            src0, dst0, rows0, src1, dst1, rows1, acc, sem0, sem1):
        c = lax.axis_index("c")
        s = lax.axis_index("s")
        my_n = jnp.where(c == 0, n0, n1)
        row0 = jnp.where(c == 0, s * (cr * n0),
                         16 * cr * n0 + s * (cr * n1))

        def fire(g, src_b, dst_b, rows_b, sem):
            base = row0 + g * cr
            pltpu.sync_copy(src_hbm.at[pl.ds(base, cr)], src_b)
            pltpu.sync_copy(dst_hbm.at[pl.ds(base, cr)], dst_b)
            for j in range(cr):
                pltpu.async_copy(y_hbm.at[src_b.at[j]],
                                 rows_b.at[pl.ds(j * 128, 128)], sem)

        def drain_scatter(src_b, dst_b, rows_b, sem):
            for j in range(cr):
                pltpu.make_async_copy(y_hbm.at[src_b.at[j]],
                                      rows_b.at[pl.ds(j * 128, 128)],
                                      sem).wait()
            for j in range(cr):
                pltpu.sync_copy(rows_b.at[pl.ds(j * 128, 128)],
                                acc.at[dst_b.at[j]], add=True)

        zr = TILE_ROWS
        while zr > cr * 128:
            zr //= 2

        def zrow(r, carry):
            for j in range(d // 32):
                rows0[r, pl.ds(j * 32, 32)] = jnp.zeros((32,), jnp.bfloat16)
            return carry
        lax.fori_loop(0, zr, zrow, 0)
        nz = TILE_ROWS // zr
        for q in range(nz):
            pltpu.sync_copy(rows0.at[pl.ds(0, zr)],
                            acc.at[pl.ds(s * TILE_ROWS + q * zr, zr)])
        plsc.subcore_barrier()

        fire(0, src0, dst0, rows0, sem0)

        def body(t, carry):
            g0 = 2 * t
            fire(g0 + 1, src1, dst1, rows1, sem1)
            drain_scatter(src0, dst0, rows0, sem0)

            @pl.when(g0 + 2 < my_n)
            def _():
                fire(g0 + 2, src0, dst0, rows0, sem0)
            drain_scatter(src1, dst1, rows1, sem1)
            return carry
        lax.fori_loop(0, my_n // 2, body, 0)
        plsc.subcore_barrier()

        pltpu.sync_copy(acc.at[pl.ds(s * TILE_ROWS, TILE_ROWS)],
                        out_hbm.at[pl.ds(c * NPAD + s * TILE_ROWS, TILE_ROWS)])

    return agg


def _layer2_body(acc_ref, b1_ref, w2_ref, o_ref):
    a = acc_ref[0].astype(jnp.float32) + acc_ref[1].astype(jnp.float32)
    lane = lax.broadcasted_iota(jnp.int32, a.shape, 1)
    deg = jnp.sum(jnp.where(lane == 100, a, 0.0), axis=1, keepdims=True)
    inv = 1.0 / jnp.maximum(deg, 1.0)
    h1 = jnp.maximum(a * inv + b1_ref[...], 0.0)
    y2 = jnp.dot(h1, w2_ref[...], preferred_element_type=jnp.float32)
    lane2 = lax.broadcasted_iota(jnp.int32, y2.shape, 1)
    o_ref[...] = (y2 + jnp.where(lane2 == 20, 1.0, 0.0)).astype(jnp.bfloat16)


def _layer2(acc1, b1p, w2p):
    return pl.pallas_call(
        _layer2_body,
        grid=(NPAD // 1024,),
        in_specs=[pl.BlockSpec((2, 1024, D1P), lambda i: (0, i, 0)),
                  pl.BlockSpec((1, D1P), lambda i: (0, 0)),
                  pl.BlockSpec((D1P, D2P), lambda i: (0, 0))],
        out_specs=pl.BlockSpec((1024, D2P), lambda i: (i, 0)),
        out_shape=jax.ShapeDtypeStruct((NPAD, D2P), jnp.bfloat16),
    )(acc1, b1p, w2p)


def _readout_body(acc_ref, gid_ref, b2_ref, o_ref):
    i = pl.program_id(0)
    a = acc_ref[0].astype(jnp.float32) + acc_ref[1].astype(jnp.float32)
    lane = lax.broadcasted_iota(jnp.int32, a.shape, 1)
    deg = jnp.sum(jnp.where(lane == 20, a, 0.0), axis=1, keepdims=True)
    inv = 1.0 / jnp.maximum(deg, 1.0)
    h2 = jnp.maximum(jnp.where(lane < 20, a * inv, 0.0) + b2_ref[...], 0.0)
    gid = gid_ref[0]
    bidx = lax.broadcasted_iota(jnp.int32, (B, 1024), 0)
    oh = (bidx == gid).astype(jnp.float32)
    ps = jnp.dot(oh, h2, preferred_element_type=jnp.float32)

    @pl.when(i == 0)
    def _():
        o_ref[...] = ps

    @pl.when(i != 0)
    def _():
        o_ref[...] += ps


def _readout(acc2, gidp, b2p):
    return pl.pallas_call(
        _readout_body,
        grid=(NPAD // 1024,),
        in_specs=[pl.BlockSpec((2, 1024, D2P), lambda i: (0, i, 0)),
                  pl.BlockSpec((1, 1, 1024), lambda i: (i, 0, 0)),
                  pl.BlockSpec((1, D2P), lambda i: (0, 0))],
        out_specs=pl.BlockSpec((B, D2P), lambda i: (0, 0)),
        out_shape=jax.ShapeDtypeStruct((B, D2P), jnp.float32),
    )(acc2, gidp, b2p)


def _final_body(hgs_ref, d2_ref, wg2_ref, bg2_ref, wtm_ref, wtl_ref, bf1_ref,
                wf2_ref, bf2_ref, wf3_ref, bf3_ref, g1_ref, be1_ref,
                g2_ref, be2_ref, o_ref):
    hgs = hgs_ref[...]
    lane = lax.broadcasted_iota(jnp.int32, hgs.shape, 1)
    cnt = jnp.sum(jnp.where(lane == 20, hgs, 0.0), axis=1, keepdims=True)
    hg = hgs * (1.0 / jnp.maximum(cnt, 1.0))
    z = jnp.dot(hg, wg2_ref[...], preferred_element_type=jnp.float32)
    g2v = 1.0 / (1.0 + jnp.exp(-(z + bg2_ref[...])))
    v2 = g2v * d2_ref[...]
    u = jnp.dot(v2, wtm_ref[...], preferred_element_type=jnp.float32)
    u = u + wtl_ref[...]
    out1 = u[:, 20 * 64:21 * 64]
    for i in range(20):
        coli = jnp.sum(jnp.where(lane == i, hg, 0.0), axis=1, keepdims=True)
        out1 = out1 + coli * u[:, i * 64:(i + 1) * 64]
    out1 = out1 + bf1_ref[...]
    m1 = jnp.mean(out1, axis=0, keepdims=True)
    v1 = jnp.mean((out1 - m1) ** 2, axis=0, keepdims=True)
    z1 = (out1 - m1) * lax.rsqrt(v1 + 1e-5) * g1_ref[...] + be1_ref[...]
    z1 = jnp.maximum(z1, 0.0)
    t2 = jnp.dot(z1, wf2_ref[...], preferred_element_type=jnp.float32)
    t2 = t2 + bf2_ref[...]
    m2 = jnp.mean(t2, axis=0, keepdims=True)
    vv2 = jnp.mean((t2 - m2) ** 2, axis=0, keepdims=True)
    z2 = (t2 - m2) * lax.rsqrt(vv2 + 1e-5) * g2_ref[...] + be2_ref[...]
    z2 = jnp.maximum(z2, 0.0)
    o_ref[...] = jnp.dot(z2, wf3_ref[...],
                         preferred_element_type=jnp.float32) + bf3_ref[...]


def _final(hgs, desc_2d, wg2p, bg2, wtm, wtl, bf1, wf2, bf2, wf3, bf3,
           gamma1, beta1, gamma2, beta2):
    full = lambda s: pl.BlockSpec(s, lambda: (0,) * len(s))
    return pl.pallas_call(
        _final_body,
        in_specs=[full((B, D2P)), full((B, 200)), full((D2P, 200)),
                  full((1, 200)), full((200, 21 * 64)), full((1, 21 * 64)),
                  full((1, 64)), full((64, 16)), full((1, 16)),
                  full((16, 1)), full((1, 1)), full((1, 64)), full((1, 64)),
                  full((1, 16)), full((1, 16))],
        out_specs=full((B, 1)),
        out_shape=jax.ShapeDtypeStruct((B, 1), jnp.float32),
    )(hgs, desc_2d, wg2p, bg2, wtm, wtl, bf1, wf2, bf2, wf3, bf3,
      gamma1, beta1, gamma2, beta2)


def kernel(x, edge_index, graph_ids, desc_2d, desc_3d, W1, b1, W2, b2,
           Wg2, bg2, Wg3, bg3, Wf1, bf1, Wf2, bf2, Wf3, bf3,
           gamma1, beta1, gamma2, beta2):
    f32 = jnp.float32
    i32 = jnp.int32

    w1p = jnp.pad(W1, ((0, 0), (0, D1P - 100)))
    b1p = jnp.pad(b1, (0, D1P - 100)).reshape(1, D1P)
    w2p = jnp.pad(W2, ((0, D1P - 100), (0, D2P - 20)))
    b2p = jnp.concatenate(
        [b2, jnp.ones((1,), f32), jnp.zeros((D2P - 21,), f32)]).reshape(1, D2P)
    src2 = jnp.concatenate(
        [edge_index[0].astype(i32), jnp.zeros((EPAD - E,), i32)]).reshape(EROWS, 128)
    dst2 = jnp.concatenate(
        [edge_index[1].astype(i32), jnp.full((EPAD - E,), N, i32)]).reshape(EROWS, 128)
    gidp = jnp.concatenate(
        [graph_ids.astype(i32), jnp.full((NPAD - N,), B, i32)]
    ).reshape(NPAD // 1024, 1, 1024)
    wg2p = jnp.pad(Wg2, ((0, D2P - 20), (0, 0)))
    wt = Wf1.reshape(21, 201, 64).transpose(1, 0, 2).reshape(201, 21 * 64)

    y1 = _mm1(x, w1p)
    acc1 = _make_agg(D1P, 2, 22, 58)(y1, src2, dst2).reshape(2, NPAD, D1P)
    y2 = _layer2(acc1, b1p, w2p)
    acc2 = _make_agg(D2P, 8, 6, 14)(y2, src2, dst2).reshape(2, NPAD, D2P)
    hgs = _readout(acc2, gidp, b2p)
    return _final(hgs, desc_2d, wg2p, bg2.reshape(1, 200), wt[:200],
                  wt[200:201], bf1.reshape(1, 64), Wf2, bf2.reshape(1, 16),
                  Wf3, bf3.reshape(1, 1), gamma1.reshape(1, 64),
                  beta1.reshape(1, 64), gamma2.reshape(1, 16),
                  beta2.reshape(1, 16))


# speedup vs baseline: 1.1109x; 1.1109x over previous
import functools

import jax
import jax.numpy as jnp
from jax import lax
from jax.experimental import pallas as pl
from jax.experimental.pallas import tpu as pltpu
from jax.experimental.pallas import tpu_sc as plsc

N = 10000
E = 320000
B = 256
NPAD = 10240
EPAD = 327680
D1P = 128
D2P = 32
EROWS = EPAD // 128
ROWS_PER_W = EROWS // 32
CHUNK_ROWS = 2
NCHUNK = ROWS_PER_W // CHUNK_ROWS
TILE_ROWS = NPAD // 16


def _mm1_body(x_ref, w_ref, o_ref):
    y = jnp.dot(x_ref[...], w_ref[...], preferred_element_type=jnp.float32)
    lane = lax.broadcasted_iota(jnp.int32, y.shape, 1)
    o_ref[...] = (y + jnp.where(lane == 100, 1.0, 0.0)).astype(jnp.bfloat16)


def _mm1(x, w1p):
    return pl.pallas_call(
        _mm1_body,
        grid=(10,),
        in_specs=[pl.BlockSpec((1000, 128), lambda i: (i, 0)),
                  pl.BlockSpec((128, D1P), lambda i: (0, 0))],
        out_specs=pl.BlockSpec((1000, D1P), lambda i: (i, 0)),
        out_shape=jax.ShapeDtypeStruct((N, D1P), jnp.bfloat16),
    )(x, w1p)


def _make_agg(d, cr, n0, n1):
    assert (n0 + n1) * 16 * cr == EROWS and n0 % 2 == 0 and n1 % 2 == 0
    mesh = plsc.VectorSubcoreMesh(core_axis_name="c", subcore_axis_name="s")

    @functools.partial(
        pl.kernel,
        mesh=mesh,
        compiler_params=pltpu.CompilerParams(use_tc_tiling_on_sc=False),
        out_type=jax.ShapeDtypeStruct((2 * NPAD, d), jnp.bfloat16),
        scratch_types=[
            pltpu.VMEM((cr, 128), jnp.int32),
            pltpu.VMEM((cr, 128), jnp.int32),
            pltpu.VMEM((cr * 128, d), jnp.bfloat16),
            pltpu.VMEM((cr, 128), jnp.int32),
            pltpu.VMEM((cr, 128), jnp.int32),
            pltpu.VMEM((cr * 128, d), jnp.bfloat16),
            pltpu.VMEM_SHARED((NPAD, d), jnp.bfloat16),
            pltpu.SemaphoreType.DMA,
            pltpu.SemaphoreType.DMA,
        ],
    )
    def agg(y_hbm, src_hbm, dst_hbm, out_hbm,
            src0, dst0, rows0, src1, dst1, rows1, acc, sem0, sem1):
        c = lax.axis_index("c")
        s = lax.axis_index("s")
        my_n = jnp.where(c == 0, n0, n1)
        row0 = jnp.where(c == 0, s * (cr * n0),
                         16 * cr * n0 + s * (cr * n1))

        def fire(g, src_b, dst_b, rows_b, sem):
            base = row0 + g * cr
            pltpu.sync_copy(src_hbm.at[pl.ds(base, cr)], src_b)
            pltpu.sync_copy(dst_hbm.at[pl.ds(base, cr)], dst_b)
            for j in range(cr):
                pltpu.async_copy(y_hbm.at[src_b.at[j]],
                                 rows_b.at[pl.ds(j * 128, 128)], sem)

        def drain_scatter(src_b, dst_b, rows_b, sem):
            for j in range(cr):
                pltpu.make_async_copy(y_hbm.at[src_b.at[j]],
                                      rows_b.at[pl.ds(j * 128, 128)],
                                      sem).wait()
            for j in range(cr):
                pltpu.sync_copy(rows_b.at[pl.ds(j * 128, 128)],
                                acc.at[dst_b.at[j]], add=True)

        zr = TILE_ROWS
        while zr > cr * 128:
            zr //= 2

        def zrow(r, carry):
            for j in range(d // 32):
                rows0[r, pl.ds(j * 32, 32)] = jnp.zeros((32,), jnp.bfloat16)
            return carry
        lax.fori_loop(0, zr, zrow, 0)
        nz = TILE_ROWS // zr
        for q in range(nz):
            pltpu.sync_copy(rows0.at[pl.ds(0, zr)],
                            acc.at[pl.ds(s * TILE_ROWS + q * zr, zr)])
        plsc.subcore_barrier()

        fire(0, src0, dst0, rows0, sem0)

        def body(t, carry):
            g0 = 2 * t
            fire(g0 + 1, src1, dst1, rows1, sem1)
            drain_scatter(src0, dst0, rows0, sem0)

            @pl.when(g0 + 2 < my_n)
            def _():
                fire(g0 + 2, src0, dst0, rows0, sem0)
            drain_scatter(src1, dst1, rows1, sem1)
            return carry
        lax.fori_loop(0, my_n // 2, body, 0)
        plsc.subcore_barrier()

        pltpu.sync_copy(acc.at[pl.ds(s * TILE_ROWS, TILE_ROWS)],
                        out_hbm.at[pl.ds(c * NPAD + s * TILE_ROWS, TILE_ROWS)])

    return agg


def _layer2_body(acc_ref, b1_ref, w2_ref, o_ref):
    a = acc_ref[0].astype(jnp.float32) + acc_ref[1].astype(jnp.float32)
    lane = lax.broadcasted_iota(jnp.int32, a.shape, 1)
    deg = jnp.sum(jnp.where(lane == 100, a, 0.0), axis=1, keepdims=True)
    inv = 1.0 / jnp.maximum(deg, 1.0)
    h1 = jnp.maximum(a * inv + b1_ref[...], 0.0)
    y2 = jnp.dot(h1, w2_ref[...], preferred_element_type=jnp.float32)
    lane2 = lax.broadcasted_iota(jnp.int32, y2.shape, 1)
    o_ref[...] = (y2 + jnp.where(lane2 == 20, 1.0, 0.0)).astype(jnp.bfloat16)


def _layer2(acc1, b1p, w2p):
    return pl.pallas_call(
        _layer2_body,
        grid=(NPAD // 1024,),
        in_specs=[pl.BlockSpec((2, 1024, D1P), lambda i: (0, i, 0)),
                  pl.BlockSpec((1, D1P), lambda i: (0, 0)),
                  pl.BlockSpec((D1P, D2P), lambda i: (0, 0))],
        out_specs=pl.BlockSpec((1024, D2P), lambda i: (i, 0)),
        out_shape=jax.ShapeDtypeStruct((NPAD, D2P), jnp.bfloat16),
    )(acc1, b1p, w2p)


def _readout_body(acc_ref, gid_ref, b2_ref, o_ref):
    i = pl.program_id(0)
    a = acc_ref[0].astype(jnp.float32) + acc_ref[1].astype(jnp.float32)
    lane = lax.broadcasted_iota(jnp.int32, a.shape, 1)
    deg = jnp.sum(jnp.where(lane == 20, a, 0.0), axis=1, keepdims=True)
    inv = 1.0 / jnp.maximum(deg, 1.0)
    h2 = jnp.maximum(jnp.where(lane < 20, a * inv, 0.0) + b2_ref[...], 0.0)
    gid = gid_ref[0]
    bidx = lax.broadcasted_iota(jnp.int32, (B, 1024), 0)
    oh = (bidx == gid).astype(jnp.float32)
    ps = jnp.dot(oh, h2, preferred_element_type=jnp.float32)

    @pl.when(i == 0)
    def _():
        o_ref[...] = ps

    @pl.when(i != 0)
    def _():
        o_ref[...] += ps


def _readout(acc2, gidp, b2p):
    return pl.pallas_call(
        _readout_body,
        grid=(NPAD // 1024,),
        in_specs=[pl.BlockSpec((2, 1024, D2P), lambda i: (0, i, 0)),
                  pl.BlockSpec((1, 1, 1024), lambda i: (i, 0, 0)),
                  pl.BlockSpec((1, D2P), lambda i: (0, 0))],
        out_specs=pl.BlockSpec((B, D2P), lambda i: (0, 0)),
        out_shape=jax.ShapeDtypeStruct((B, D2P), jnp.float32),
    )(acc2, gidp, b2p)


def _final_body(hgs_ref, d2_ref, wg2_ref, bg2_ref, wtm_ref, wtl_ref, bf1_ref,
                wf2_ref, bf2_ref, wf3_ref, bf3_ref, g1_ref, be1_ref,
                g2_ref, be2_ref, o_ref):
    hgs = hgs_ref[...]
    lane = lax.broadcasted_iota(jnp.int32, hgs.shape, 1)
    cnt = jnp.sum(jnp.where(lane == 20, hgs, 0.0), axis=1, keepdims=True)
    hg = hgs * (1.0 / jnp.maximum(cnt, 1.0))
    z = jnp.dot(hg, wg2_ref[...], preferred_element_type=jnp.float32)
    g2v = 1.0 / (1.0 + jnp.exp(-(z + bg2_ref[...])))
    v2 = g2v * d2_ref[...]
    u = jnp.dot(v2, wtm_ref[...], preferred_element_type=jnp.float32)
    u = u + wtl_ref[...]
    out1 = u[:, 20 * 64:21 * 64]
    for i in range(20):
        coli = jnp.sum(jnp.where(lane == i, hg, 0.0), axis=1, keepdims=True)
        out1 = out1 + coli * u[:, i * 64:(i + 1) * 64]
    out1 = out1 + bf1_ref[...]
    m1 = jnp.mean(out1, axis=0, keepdims=True)
    v1 = jnp.mean((out1 - m1) ** 2, axis=0, keepdims=True)
    z1 = (out1 - m1) * lax.rsqrt(v1 + 1e-5) * g1_ref[...] + be1_ref[...]
    z1 = jnp.maximum(z1, 0.0)
    t2 = jnp.dot(z1, wf2_ref[...], preferred_element_type=jnp.float32)
    t2 = t2 + bf2_ref[...]
    m2 = jnp.mean(t2, axis=0, keepdims=True)
    vv2 = jnp.mean((t2 - m2) ** 2, axis=0, keepdims=True)
    z2 = (t2 - m2) * lax.rsqrt(vv2 + 1e-5) * g2_ref[...] + be2_ref[...]
    z2 = jnp.maximum(z2, 0.0)
    o_ref[...] = jnp.dot(z2, wf3_ref[...],
                         preferred_element_type=jnp.float32) + bf3_ref[...]


def _final(hgs, desc_2d, wg2p, bg2, wtm, wtl, bf1, wf2, bf2, wf3, bf3,
           gamma1, beta1, gamma2, beta2):
    full = lambda s: pl.BlockSpec(s, lambda: (0,) * len(s))
    return pl.pallas_call(
        _final_body,
        in_specs=[full((B, D2P)), full((B, 200)), full((D2P, 200)),
                  full((1, 200)), full((200, 21 * 64)), full((1, 21 * 64)),
                  full((1, 64)), full((64, 16)), full((1, 16)),
                  full((16, 1)), full((1, 1)), full((1, 64)), full((1, 64)),
                  full((1, 16)), full((1, 16))],
        out_specs=full((B, 1)),
        out_shape=jax.ShapeDtypeStruct((B, 1), jnp.float32),
    )(hgs, desc_2d, wg2p, bg2, wtm, wtl, bf1, wf2, bf2, wf3, bf3,
      gamma1, beta1, gamma2, beta2)


def kernel(x, edge_index, graph_ids, desc_2d, desc_3d, W1, b1, W2, b2,
           Wg2, bg2, Wg3, bg3, Wf1, bf1, Wf2, bf2, Wf3, bf3,
           gamma1, beta1, gamma2, beta2):
    f32 = jnp.float32
    i32 = jnp.int32

    w1p = jnp.pad(W1, ((0, 0), (0, D1P - 100)))
    b1p = jnp.pad(b1, (0, D1P - 100)).reshape(1, D1P)
    w2p = jnp.pad(W2, ((0, D1P - 100), (0, D2P - 20)))
    b2p = jnp.concatenate(
        [b2, jnp.ones((1,), f32), jnp.zeros((D2P - 21,), f32)]).reshape(1, D2P)
    src2 = jnp.concatenate(
        [edge_index[0].astype(i32), jnp.zeros((EPAD - E,), i32)]).reshape(EROWS, 128)
    dst2 = jnp.concatenate(
        [edge_index[1].astype(i32), jnp.full((EPAD - E,), N, i32)]).reshape(EROWS, 128)
    gidp = jnp.concatenate(
        [graph_ids.astype(i32), jnp.full((NPAD - N,), B, i32)]
    ).reshape(NPAD // 1024, 1, 1024)
    wg2p = jnp.pad(Wg2, ((0, D2P - 20), (0, 0)))
    wt = Wf1.reshape(21, 201, 64).transpose(1, 0, 2).reshape(201, 21 * 64)

    y1 = _mm1(x, w1p)
    acc1 = _make_agg(D1P, 2, 58, 22)(y1, src2, dst2).reshape(2, NPAD, D1P)
    y2 = _layer2(acc1, b1p, w2p)
    acc2 = _make_agg(D2P, 4, 26, 14)(y2, src2, dst2).reshape(2, NPAD, D2P)
    hgs = _readout(acc2, gidp, b2p)
    return _final(hgs, desc_2d, wg2p, bg2.reshape(1, 200), wt[:200],
                  wt[200:201], bf1.reshape(1, 64), Wf2, bf2.reshape(1, 16),
                  Wf3, bf3.reshape(1, 1), gamma1.reshape(1, 64),
                  beta1.reshape(1, 64), gamma2.reshape(1, 16),
                  beta2.reshape(1, 16))
